# baseline (speedup 1.0000x reference)
import functools

import jax
import jax.numpy as jnp
from jax import lax
from jax.experimental import pallas as pl
from jax.experimental.pallas import tpu as pltpu
from jax.experimental.pallas import tpu_sc as plsc

EMB = 64
NB = 8


def _k1_body(x_ref, rbf_ref, wkj_ref, bkj_ref, wrbf_ref, out_ref):
    xk = jnp.dot(x_ref[...], wkj_ref[...], preferred_element_type=jnp.float32)
    xk = xk + bkj_ref[...]
    g = jnp.dot(rbf_ref[...], wrbf_ref[...], preferred_element_type=jnp.float32)
    res = xk * g
    out_ref[...] = jnp.concatenate([res, jnp.zeros_like(res)], axis=1)


def _k1(x, rbf, w_kj, b_kj, w_rbf):
    e, emb = x.shape
    nr = rbf.shape[1]
    et = 2000
    grid = (pl.cdiv(e, et),)
    return pl.pallas_call(
        _k1_body,
        grid=grid,
        in_specs=[
            pl.BlockSpec((et, emb), lambda i: (i, 0)),
            pl.BlockSpec((et, nr), lambda i: (i, 0)),
            pl.BlockSpec((emb, emb), lambda i: (0, 0)),
            pl.BlockSpec((1, emb), lambda i: (0, 0)),
            pl.BlockSpec((nr, emb), lambda i: (0, 0)),
        ],
        out_specs=pl.BlockSpec((et, 2 * emb), lambda i: (i, 0)),
        out_shape=jax.ShapeDtypeStruct((e, 2 * emb), jnp.float32),
    )(x, rbf, w_kj, b_kj.reshape(1, emb), w_rbf)


def _k3_body(sbf_ref, xg_ref, wsbf_ref, wt_ref, out_ref):
    s2 = jnp.dot(sbf_ref[...], wsbf_ref[...], preferred_element_type=jnp.float32)
    xg = xg_ref[:, :EMB]
    acc = jnp.zeros_like(xg)
    for j in range(NB):
        acc = acc + jnp.dot(
            s2[:, j : j + 1] * xg, wt_ref[j], preferred_element_type=jnp.float32
        )
    out_ref[...] = jnp.concatenate([acc, jnp.zeros_like(acc)], axis=1)


def _k3(sbf, xg, w_sbf, wt):
    t, nsr = sbf.shape
    emb = EMB
    tt = 2000
    grid = (pl.cdiv(t, tt),)
    return pl.pallas_call(
        _k3_body,
        grid=grid,
        in_specs=[
            pl.BlockSpec((tt, nsr), lambda i: (i, 0)),
            pl.BlockSpec((tt, 2 * emb), lambda i: (i, 0)),
            pl.BlockSpec((nsr, NB), lambda i: (0, 0)),
            pl.BlockSpec((NB, emb, emb), lambda i: (0, 0, 0)),
        ],
        out_specs=pl.BlockSpec((tt, 2 * emb), lambda i: (i, 0)),
        out_shape=jax.ShapeDtypeStruct((t, 2 * emb), jnp.float32),
    )(sbf, xg, w_sbf, wt)


def _k5_body(x_ref, red_ref, ws_ref, bs_ref, out_ref):
    def mm(a, j):
        return jnp.dot(a, ws_ref[j], preferred_element_type=jnp.float32) + bs_ref[j]

    x_t = x_ref[...]
    x2 = mm(x_t, 0) + red_ref[:, :EMB]
    x2 = x2 + mm(mm(x2, 1), 2)
    x2 = mm(x2, 3)
    xo = x_t + x2
    xo = xo + mm(mm(xo, 4), 5)
    xo = xo + mm(mm(xo, 6), 7)
    out_ref[...] = xo


def _k5(x, red, ws, bs):
    e, emb = x.shape
    et = 2000
    grid = (pl.cdiv(e, et),)
    return pl.pallas_call(
        _k5_body,
        grid=grid,
        in_specs=[
            pl.BlockSpec((et, emb), lambda i: (i, 0)),
            pl.BlockSpec((et, 2 * emb), lambda i: (i, 0)),
            pl.BlockSpec((8, emb, emb), lambda i: (0, 0, 0)),
            pl.BlockSpec((8, 1, emb), lambda i: (0, 0, 0)),
        ],
        out_specs=pl.BlockSpec((et, emb), lambda i: (i, 0)),
        out_shape=jax.ShapeDtypeStruct((e, emb), jnp.float32),
    )(x, red, ws, bs)


_GCH = 512
_GSUB = 128


def _k2_gather(table, ids):
    t = ids.shape[0]
    emb = table.shape[1]
    nch = t // _GCH
    assert t % _GCH == 0
    nw = 32
    mesh = plsc.VectorSubcoreMesh(core_axis_name="c", subcore_axis_name="s")

    @functools.partial(
        pl.kernel,
        out_type=jax.ShapeDtypeStruct((t, emb), jnp.float32),
        mesh=mesh,
        scratch_types=[
            pltpu.VMEM((_GCH,), jnp.int32),
            pltpu.VMEM((_GCH, emb), jnp.float32),
            pltpu.SemaphoreType.DMA,
        ],
    )
    def k(table_hbm, idx_hbm, out_hbm, idx_v, rows_v, sem):
        wid = lax.axis_index("s") * 2 + lax.axis_index("c")

        def body(i, _):
            ch = wid + i * nw

            @pl.when(ch < nch)
            def _():
                base = ch * _GCH
                pltpu.sync_copy(idx_hbm.at[pl.ds(base, _GCH)], idx_v)
                descs = []
                for j in range(_GCH // _GSUB):
                    descs.append(
                        pltpu.async_copy(
                            table_hbm.at[idx_v.at[pl.ds(j * _GSUB, _GSUB)]],
                            rows_v.at[pl.ds(j * _GSUB, _GSUB)],
                            sem,
                        )
                    )
                for d in descs:
                    d.wait()
                pltpu.sync_copy(rows_v, out_hbm.at[pl.ds(base, _GCH)])

            return 0

        lax.fori_loop(0, pl.cdiv(nch, nw), body, 0, unroll=False)

    return k(table, ids)


_SREAL = 10240
_SROWS = 10368
_TRASH = 10240
_SCH = 256
_SSUB = 128


def _k4_scatter(out_w, ids, e):
    t = ids.shape[0]
    emb = EMB
    npass = pl.cdiv(e, 2 * _SREAL)
    nch = t // _SCH
    assert t % _SCH == 0
    nst = 16
    zrows = _SROWS // 16
    drows = _SREAL // 16
    zsub = 64
    dsub = 128
    mesh = plsc.VectorSubcoreMesh(core_axis_name="c", subcore_axis_name="s")

    @functools.partial(
        pl.kernel,
        out_type=jax.ShapeDtypeStruct((e, 2 * emb), jnp.float32),
        mesh=mesh,
        scratch_types=[
            pltpu.VMEM((_SCH,), jnp.int32),
            pltpu.VMEM((_SCH, 2 * emb), jnp.float32),
            pltpu.VMEM((_SCH // _SSUB, _SSUB), jnp.int32),
            pltpu.VMEM((zsub, 2 * emb), jnp.float32),
            pltpu.VMEM_SHARED((_SROWS, 2 * emb), jnp.float32),
            pltpu.SemaphoreType.DMA,
        ],
    )
    def k(rows_hbm, idx_hbm, out_hbm, idx_v, rows_v, offs_v, zbuf, acc, sem):
        c = lax.axis_index("c")
        s = lax.axis_index("s")

        for r in range(zsub):
            for q in range(2 * emb // 16):
                zbuf[r, pl.ds(q * 16, 16)] = jnp.zeros((16,), jnp.float32)

        def do_pass(p, _):
            lo = p * (2 * _SREAL) + c * _SREAL

            zbase = s * zrows
            for zi in range(pl.cdiv(zrows, zsub)):
                n = min(zsub, zrows - zi * zsub)
                pltpu.sync_copy(
                    zbuf.at[pl.ds(0, n)],
                    acc.at[pl.ds(zbase + zi * zsub, n)],
                )
            plsc.subcore_barrier()

            def chunk(i, _):
                ch = s + i * nst

                @pl.when(ch < nch)
                def _():
                    base = ch * _SCH
                    pltpu.sync_copy(idx_hbm.at[pl.ds(base, _SCH)], idx_v)
                    rd = pltpu.async_copy(
                        rows_hbm.at[pl.ds(base, _SCH)], rows_v, sem
                    )
                    for j in range(_SCH // _SSUB):
                        for q in range(_SSUB // 16):
                            idv = idx_v[pl.ds(j * _SSUB + q * 16, 16)]
                            m = (idv >= lo) & (idv < lo + _SREAL)
                            off = jnp.where(m, idv - lo, _TRASH)
                            offs_v[j, pl.ds(q * 16, 16)] = off
                    rd.wait()
                    for j in range(_SCH // _SSUB):
                        pltpu.sync_copy(
                            rows_v.at[pl.ds(j * _SSUB, _SSUB)],
                            acc.at[offs_v.at[j]],
                            add=True,
                        )

                return 0

            lax.fori_loop(0, pl.cdiv(nch, nst), chunk, 0, unroll=False)
            plsc.subcore_barrier()

            rbase = s * drows

            @pl.when(lo + rbase < e)
            def _():
                for di in range(drows // dsub):
                    off = rbase + di * dsub
                    pltpu.sync_copy(acc.at[pl.ds(off, dsub)],
                                    rows_v.at[pl.ds(0, dsub)])
                    pltpu.sync_copy(rows_v.at[pl.ds(0, dsub)],
                                    out_hbm.at[pl.ds(lo + off, dsub)])

            plsc.subcore_barrier()
            return 0

        lax.fori_loop(0, npass, do_pass, 0, unroll=False)

    return k(out_w, ids)


def kernel(x, rbf, sbf, id_expand_kj, id_reduce_ji, W_ji, b_ji, W_kj, b_kj,
           W_rbf, W_sbf, W_bilin, bs0_W1, bs0_b1, bs0_W2, bs0_b2, Wf, bf,
           as0_W1, as0_b1, as0_W2, as0_b2, as1_W1, as1_b1, as1_W2, as1_b2):
    e = x.shape[0]
    emb = x.shape[1]

    x_kj = _k1(x, rbf, W_kj, b_kj, W_rbf)
    xg = _k2_gather(x_kj, id_expand_kj)

    wt = jnp.transpose(W_bilin, (1, 2, 0))
    out_w = _k3(sbf, xg, W_sbf, wt)

    red = _k4_scatter(out_w, id_reduce_ji, e)

    ws = jnp.stack([W_ji, bs0_W1, bs0_W2, Wf, as0_W1, as0_W2, as1_W1, as1_W2])
    bs = jnp.stack([b_ji, bs0_b1, bs0_b2, bf, as0_b1, as0_b2, as1_b1,
                    as1_b2]).reshape(8, 1, emb)
    return _k5(x, red, ws, bs)


# speedup vs baseline: 3.4330x; 3.4330x over previous
import functools

import jax
import jax.numpy as jnp
from jax import lax
from jax.experimental import pallas as pl
from jax.experimental.pallas import tpu as pltpu
from jax.experimental.pallas import tpu_sc as plsc

EMB = 64
NB = 8


def _k1_body(x_ref, rbf_ref, wkj_ref, bkj_ref, wrbf_ref, out_ref):
    xk = jnp.dot(x_ref[...], wkj_ref[...], preferred_element_type=jnp.float32)
    xk = xk + bkj_ref[...]
    g = jnp.dot(rbf_ref[...], wrbf_ref[...], preferred_element_type=jnp.float32)
    res = xk * g
    out_ref[...] = jnp.concatenate([res, jnp.zeros_like(res)], axis=1)


def _k1(x, rbf, w_kj, b_kj, w_rbf):
    e, emb = x.shape
    nr = rbf.shape[1]
    et = 2000
    grid = (pl.cdiv(e, et),)
    return pl.pallas_call(
        _k1_body,
        grid=grid,
        in_specs=[
            pl.BlockSpec((et, emb), lambda i: (i, 0)),
            pl.BlockSpec((et, nr), lambda i: (i, 0)),
            pl.BlockSpec((emb, emb), lambda i: (0, 0)),
            pl.BlockSpec((1, emb), lambda i: (0, 0)),
            pl.BlockSpec((nr, emb), lambda i: (0, 0)),
        ],
        out_specs=pl.BlockSpec((et, 2 * emb), lambda i: (i, 0)),
        out_shape=jax.ShapeDtypeStruct((e, 2 * emb), jnp.float32),
    )(x, rbf, w_kj, b_kj.reshape(1, emb), w_rbf)


def _k3_body(sbf_ref, xg_ref, wsbf_ref, wt_ref, out_ref):
    s2 = jnp.dot(sbf_ref[...], wsbf_ref[...], preferred_element_type=jnp.float32)
    xg = xg_ref[:, :EMB]
    acc = jnp.zeros_like(xg)
    for j in range(NB):
        acc = acc + jnp.dot(
            s2[:, j : j + 1] * xg, wt_ref[j], preferred_element_type=jnp.float32
        )
    out_ref[...] = jnp.concatenate([acc, jnp.zeros_like(acc)], axis=1)


def _k3(sbf, xg, w_sbf, wt):
    t, nsr = sbf.shape
    emb = EMB
    tt = 2000
    grid = (pl.cdiv(t, tt),)
    return pl.pallas_call(
        _k3_body,
        grid=grid,
        in_specs=[
            pl.BlockSpec((tt, nsr), lambda i: (i, 0)),
            pl.BlockSpec((tt, 2 * emb), lambda i: (i, 0)),
            pl.BlockSpec((nsr, NB), lambda i: (0, 0)),
            pl.BlockSpec((NB, emb, emb), lambda i: (0, 0, 0)),
        ],
        out_specs=pl.BlockSpec((tt, 2 * emb), lambda i: (i, 0)),
        out_shape=jax.ShapeDtypeStruct((t, 2 * emb), jnp.float32),
    )(sbf, xg, w_sbf, wt)


def _k5_body(x_ref, red_ref, ws_ref, bs_ref, out_ref):
    def mm(a, j):
        return jnp.dot(a, ws_ref[j], preferred_element_type=jnp.float32) + bs_ref[j]

    x_t = x_ref[...]
    x2 = mm(x_t, 0) + red_ref[:, :EMB]
    x2 = x2 + mm(mm(x2, 1), 2)
    x2 = mm(x2, 3)
    xo = x_t + x2
    xo = xo + mm(mm(xo, 4), 5)
    xo = xo + mm(mm(xo, 6), 7)
    out_ref[...] = xo


def _k5(x, red, ws, bs):
    e, emb = x.shape
    et = 2000
    grid = (pl.cdiv(e, et),)
    return pl.pallas_call(
        _k5_body,
        grid=grid,
        in_specs=[
            pl.BlockSpec((et, emb), lambda i: (i, 0)),
            pl.BlockSpec((et, 2 * emb), lambda i: (i, 0)),
            pl.BlockSpec((8, emb, emb), lambda i: (0, 0, 0)),
            pl.BlockSpec((8, 1, emb), lambda i: (0, 0, 0)),
        ],
        out_specs=pl.BlockSpec((et, emb), lambda i: (i, 0)),
        out_shape=jax.ShapeDtypeStruct((e, emb), jnp.float32),
    )(x, red, ws, bs)


_GCH = 512
_GSUB = 128


def _k2_gather(table, ids):
    t = ids.shape[0]
    emb = table.shape[1]
    nch = t // _GCH
    assert t % _GCH == 0
    nw = 32
    mesh = plsc.VectorSubcoreMesh(core_axis_name="c", subcore_axis_name="s")

    @functools.partial(
        pl.kernel,
        out_type=jax.ShapeDtypeStruct((t, emb), jnp.float32),
        mesh=mesh,
        scratch_types=[
            pltpu.VMEM((_GCH,), jnp.int32),
            pltpu.VMEM((_GCH, emb), jnp.float32),
            pltpu.SemaphoreType.DMA,
        ],
    )
    def k(table_hbm, idx_hbm, out_hbm, idx_v, rows_v, sem):
        wid = lax.axis_index("s") * 2 + lax.axis_index("c")

        def body(i, _):
            ch = wid + i * nw

            @pl.when(ch < nch)
            def _():
                base = ch * _GCH
                pltpu.sync_copy(idx_hbm.at[pl.ds(base, _GCH)], idx_v)
                descs = []
                for j in range(_GCH // _GSUB):
                    descs.append(
                        pltpu.async_copy(
                            table_hbm.at[idx_v.at[pl.ds(j * _GSUB, _GSUB)]],
                            rows_v.at[pl.ds(j * _GSUB, _GSUB)],
                            sem,
                        )
                    )
                for d in descs:
                    d.wait()
                pltpu.sync_copy(rows_v, out_hbm.at[pl.ds(base, _GCH)])

            return 0

        lax.fori_loop(0, pl.cdiv(nch, nw), body, 0, unroll=False)

    return k(table, ids)


_SREAL = 10240
_SROWS = 10368
_TRASH = 10240
_FIRE = 2048
_ICH = 4000


def _k4_scatter(out_w, ids, e):
    t = ids.shape[0]
    emb = EMB
    npass = pl.cdiv(e, 2 * _SREAL)
    per_tile = t // 16
    nich = per_tile // _ICH
    nstep = _ICH // 16
    zrows = _SROWS // 16
    drows = _SREAL // 16
    mesh = plsc.VectorSubcoreMesh(core_axis_name="c", subcore_axis_name="s")

    @functools.partial(
        pl.kernel,
        out_type=jax.ShapeDtypeStruct((e, 2 * emb), jnp.float32),
        mesh=mesh,
        compiler_params=pltpu.CompilerParams(needs_layout_passes=False),
        scratch_types=[
            pltpu.VMEM((2 * _ICH,), jnp.int32),
            pltpu.VMEM((2304,), jnp.int32),
            pltpu.VMEM((2304,), jnp.int32),
            pltpu.VMEM((16, 128), jnp.int32),
            pltpu.VMEM((2 * 128, 2 * emb), jnp.float32),
            pltpu.VMEM_SHARED((_SROWS, 2 * emb), jnp.float32),
            pltpu.SemaphoreType.DMA,
            pltpu.SemaphoreType.DMA,
            pltpu.SemaphoreType.DMA,
            pltpu.SemaphoreType.DMA,
        ],
    )
    def k(rows_hbm, idx_hbm, out_hbm, idx_v, wbuf, obuf, offs2, rows_v,
          acc, gsem0, gsem1, asem, isem):
        c = lax.axis_index("c")
        s = lax.axis_index("s")

        gsems = [gsem0, gsem1]

        def _fill_offs(j):
            for i in range(8):
                offs2[j, pl.ds(i * 16, 16)] = obuf[pl.ds(j * 128 + i * 16, 16)]

        def _gather(j):
            return pltpu.async_copy(
                rows_hbm.at[wbuf.at[pl.ds(j * 128, 128)]],
                rows_v.at[pl.ds((j % 2) * 128, 128)], gsems[j % 2])

        def _wait_g(j):
            pltpu.make_async_copy(
                rows_hbm.at[wbuf.at[pl.ds(j * 128, 128)]],
                rows_v.at[pl.ds((j % 2) * 128, 128)], gsems[j % 2]).wait()

        def _add(j):
            pltpu.async_copy(rows_v.at[pl.ds((j % 2) * 128, 128)],
                             acc.at[offs2.at[j]], asem, add=True)

        def _wait_a(j):
            pltpu.make_async_copy(rows_v.at[pl.ds((j % 2) * 128, 128)],
                                  acc.at[offs2.at[j]], asem).wait()

        def fire_all(nb):
            for j in range(2):
                @pl.when(j < nb)
                def _(j=j):
                    _fill_offs(j)
                    _gather(j)
            for j in range(2, 16):
                @pl.when(j < nb)
                def _(j=j):
                    _wait_g(j - 2)
                    _add(j - 2)
                    _wait_a(j - 2)
                    _fill_offs(j)
                    _gather(j)
            for j in range(16):
                @pl.when((j >= nb - 2) & (j < nb))
                def _(j=j):
                    _wait_g(j)
                    _add(j)
                    _wait_a(j)

        def do_pass(p, _):
            lo = p * (2 * _SREAL) + c * _SREAL

            for r in range(128):
                for q in range(2 * emb // 16):
                    rows_v[r, pl.ds(q * 16, 16)] = jnp.zeros((16,), jnp.float32)
            zbase = s * zrows
            zd = []
            for zi in range(pl.cdiv(zrows, 128)):
                n = min(128, zrows - zi * 128)
                zd.append(pltpu.async_copy(
                    rows_v.at[pl.ds(0, n)],
                    acc.at[pl.ds(zbase + zi * 128, n)], gsem0))
            for d in zd:
                d.wait()
            plsc.subcore_barrier()
            seg = s * per_tile
            pltpu.async_copy(idx_hbm.at[pl.ds(seg, _ICH)],
                             idx_v.at[pl.ds(0, _ICH)], isem)

            def chunk(ki, carry):
                cnt_vec = carry
                base = s * per_tile + ki * _ICH
                ioff = (ki % 2) * _ICH
                pltpu.make_async_copy(idx_hbm.at[pl.ds(seg, _ICH)],
                                      idx_v.at[pl.ds(0, _ICH)], isem).wait()

                @pl.when(ki + 1 < nich)
                def _():
                    pltpu.async_copy(
                        idx_hbm.at[pl.ds(base + _ICH, _ICH)],
                        idx_v.at[pl.ds(((ki + 1) % 2) * _ICH, _ICH)], isem)

                def quad(qq, cnt_vec):
                    for u in range(5):
                        q = qq * 5 + u
                        idv = idx_v[pl.ds(ioff + q * 16, 16)]
                        m = (idv >= lo) & (idv < lo + _SREAL)
                        pos = cnt_vec + plsc.cumsum(m.astype(jnp.int32)) - 1
                        plsc.store_scatter(obuf, [pos], idv - lo, mask=m)
                        wv = (base + q * 16) + lax.iota(jnp.int32, 16)
                        plsc.store_scatter(wbuf, [pos], wv, mask=m)
                        cnt_vec = cnt_vec + plsc.all_reduce_population_count(m)
                    full = cnt_vec[0] >= _FIRE

                    @pl.when(full)
                    def _():
                        fire_all(16)
                        for i in range(5):
                            wt = wbuf[pl.ds(_FIRE + i * 16, 16)]
                            ot = obuf[pl.ds(_FIRE + i * 16, 16)]
                            wbuf[pl.ds(i * 16, 16)] = wt
                            obuf[pl.ds(i * 16, 16)] = ot

                    cnt_vec = jnp.where(full, cnt_vec - _FIRE, cnt_vec)
                    return cnt_vec

                assert nstep % 5 == 0
                return lax.fori_loop(0, nstep // 5, quad, cnt_vec,
                                     unroll=4)

            cnt_vec = lax.fori_loop(
                0, nich, chunk, jnp.zeros((16,), jnp.int32))
            cnt = cnt_vec[0]

            for i in range(8):
                obuf[pl.ds(cnt + i * 16, 16)] = jnp.full((16,), _TRASH,
                                                         jnp.int32)
                wbuf[pl.ds(cnt + i * 16, 16)] = jnp.zeros((16,), jnp.int32)
            nb = (cnt + 127) // 128
            fire_all(nb)
            plsc.subcore_barrier()

            rbase = s * drows

            @pl.when(lo + rbase < e)
            def _():
                nblk = drows // 128
                rds = {}
                for di in range(nblk):
                    off = rbase + di * 128
                    sl = (di % 2) * 128
                    if di >= 2:
                        rds[di - 2][1].wait()
                    rds[di] = [pltpu.async_copy(acc.at[pl.ds(off, 128)],
                                                rows_v.at[pl.ds(sl, 128)],
                                                gsem0), None]
                    rds[di][0].wait()
                    rds[di][1] = pltpu.async_copy(
                        rows_v.at[pl.ds(sl, 128)],
                        out_hbm.at[pl.ds(lo + off, 128)], asem)
                for di in range(max(0, nblk - 2), nblk):
                    rds[di][1].wait()

            plsc.subcore_barrier()
            return 0

        lax.fori_loop(0, npass, do_pass, 0, unroll=False)

    return k(out_w, ids)


def kernel(x, rbf, sbf, id_expand_kj, id_reduce_ji, W_ji, b_ji, W_kj, b_kj,
           W_rbf, W_sbf, W_bilin, bs0_W1, bs0_b1, bs0_W2, bs0_b2, Wf, bf,
           as0_W1, as0_b1, as0_W2, as0_b2, as1_W1, as1_b1, as1_W2, as1_b2):
    e = x.shape[0]
    emb = x.shape[1]

    x_kj = _k1(x, rbf, W_kj, b_kj, W_rbf)
    xg = _k2_gather(x_kj, id_expand_kj)

    wt = jnp.transpose(W_bilin, (1, 2, 0))
    out_w = _k3(sbf, xg, W_sbf, wt)

    red = _k4_scatter(out_w, id_reduce_ji, e)

    ws = jnp.stack([W_ji, bs0_W1, bs0_W2, Wf, as0_W1, as0_W2, as1_W1, as1_W2])
    bs = jnp.stack([b_ji, bs0_b1, bs0_b2, bf, as0_b1, as0_b2, as1_b1,
                    as1_b2]).reshape(8, 1, emb)
    return _k5(x, red, ws, bs)
